# baseline (speedup 1.0000x reference)
import functools

import jax
import jax.numpy as jnp
import numpy as np
from jax import lax
from jax.experimental import pallas as pl
from jax.experimental.pallas import tpu as pltpu
from jax.experimental.pallas import tpu_sc as plsc

_B, _H, _D, _V = 16384, 50, 64, 1000000
_BR = 4096
_NB = 64
_Q = _BR * _NB
_VP = 4 * _Q

_NC, _NS, _L = 2, 16, 16
_NW = _NC * _NS
_BPW = _B // _NW
_C = 8
_CW = _C * _H
_NCH = _BPW // _C
_NBUF = 4
_INV = 1.0 / _H
_WIDX = _BPW * _H

_DEAL = np.concatenate(
    [np.arange(0, 16), np.arange(32, 48), np.arange(16, 32), np.arange(48, 64)]
)
_INV_DEAL = np.argsort(_DEAL)

_LAST_BLK = (_V + _BR - 1) // _BR - 1


def _pack_body(x0_ref, x1_ref, x2_ref, x3_ref, out_ref):
    x = jnp.concatenate(
        [x0_ref[...], x1_ref[...], x2_ref[...], x3_ref[...]], axis=0
    )
    y = jnp.swapaxes(x, 0, 1)
    u = lax.bitcast_convert_type(y, jnp.uint32)
    bits = (u + 0x7FFF + ((u >> 16) & 1)) >> 16
    words = []
    for q in range(4):
        lo = bits[:, q * _D: q * _D + 32]
        hi = bits[:, q * _D + 32: (q + 1) * _D]
        words.append(lo | (hi << 16))
    out_ref[...] = jnp.concatenate(words, axis=1)


def _pack(tab_t):
    specs = []
    for q in range(4):
        specs.append(
            pl.BlockSpec(
                (_D, _BR),
                functools.partial(
                    lambda g, q: (0, jnp.minimum(g + q * _NB, _LAST_BLK)), q=q
                ),
            )
        )
    return pl.pallas_call(
        _pack_body,
        out_shape=jax.ShapeDtypeStruct((_Q, 128), jnp.uint32),
        grid=(_NB,),
        in_specs=specs,
        out_specs=pl.BlockSpec((_BR, 128), lambda g: (g, 0)),
    )(tab_t, tab_t, tab_t, tab_t)


def _body(xs_hbm, tab_hbm, out_hbm, xs_v, rows_bufs, out_v, sems):
    wid = lax.axis_index("s") * _NC + lax.axis_index("c")
    base = wid * _BPW

    pltpu.sync_copy(xs_hbm.at[pl.ds(base * _H, _WIDX)], xs_v)

    @pl.loop(0, _WIDX, step=_L)
    def _remap(k):
        v = xs_v[pl.ds(k, _L)]
        xs_v[pl.ds(k, _L)] = ((v & (_Q - 1)) << 2) | jax.lax.shift_right_logical(v, 18)

    for b in range(_NBUF):
        pltpu.async_copy(
            tab_hbm.at[xs_v.at[pl.ds(b * _CW, _CW)]], rows_bufs[b], sems[b]
        )

    @pl.loop(0, _NCH, step=_NBUF)
    def _chunks(ci):
        for b in range(_NBUF):
            rows = rows_bufs[b]
            sem = sems[b]
            cur = ci + b
            pltpu.make_async_copy(
                tab_hbm.at[xs_v.at[pl.ds(cur * _CW, _CW)]], rows, sem
            ).wait()

            @pl.loop(0, _C)
            def _items(i, rows=rows, cur=cur):
                rowbase = i * _H
                z = jnp.zeros((_L,), jnp.float32)

                @pl.loop(0, _H, init_carry=(z, z, z, z), unroll=5)
                def _acc(j, carry, rows=rows, rowbase=rowbase):
                    a0, a1, a2, a3 = carry
                    rr = rowbase + j
                    lo = plsc.bitcast(rows[rr, pl.ds(0, _L)], jnp.bfloat16)
                    hi = plsc.bitcast(rows[rr, pl.ds(_L, _L)], jnp.bfloat16)
                    e0, o0 = plsc.unpack(lo, format=plsc.PackFormat.INTERLEAVED)
                    e1, o1 = plsc.unpack(hi, format=plsc.PackFormat.INTERLEAVED)
                    return (a0 + e0, a1 + o0, a2 + e1, a3 + o1)

                a0, a1, a2, a3 = _acc
                r0 = cur * _C + i
                out_v[r0, pl.ds(0, _L)] = a0 * _INV
                out_v[r0, pl.ds(_L, _L)] = a1 * _INV
                out_v[r0, pl.ds(2 * _L, _L)] = a2 * _INV
                out_v[r0, pl.ds(3 * _L, _L)] = a3 * _INV

            nxt = cur + _NBUF

            @pl.when(nxt < _NCH)
            def _fire(rows=rows, sem=sem, nxt=nxt):
                pltpu.async_copy(
                    tab_hbm.at[xs_v.at[pl.ds(nxt * _CW, _CW)]], rows, sem
                )

    pltpu.sync_copy(out_v, out_hbm.at[pl.ds(base, _BPW)])


@functools.cache
def _make_sc_kernel():
    mesh = plsc.VectorSubcoreMesh(
        core_axis_name="c", subcore_axis_name="s",
        num_cores=_NC, num_subcores=_NS,
    )

    def body(xs_hbm, tab_hbm, out_hbm, xs_v,
             r0, r1, r2, r3, out_v, s0, s1, s2, s3):
        _body(xs_hbm, tab_hbm, out_hbm, xs_v, (r0, r1, r2, r3), out_v,
              (s0, s1, s2, s3))

    return pl.kernel(
        body,
        out_type=jax.ShapeDtypeStruct((_B, _D), jnp.float32),
        mesh=mesh,
        scratch_types=[
            pltpu.VMEM((_WIDX,), jnp.int32),
            pltpu.VMEM((_CW, 2 * _L), jnp.uint32),
            pltpu.VMEM((_CW, 2 * _L), jnp.uint32),
            pltpu.VMEM((_CW, 2 * _L), jnp.uint32),
            pltpu.VMEM((_CW, 2 * _L), jnp.uint32),
            pltpu.VMEM((_BPW, _D), jnp.float32),
            pltpu.SemaphoreType.DMA,
            pltpu.SemaphoreType.DMA,
            pltpu.SemaphoreType.DMA,
            pltpu.SemaphoreType.DMA,
        ],
        compiler_params=pltpu.CompilerParams(use_tc_tiling_on_sc=False, needs_layout_passes=False),
    )


def kernel(xs, table):
    xs_flat = xs.reshape(-1).astype(jnp.int32)
    tab_packed = _pack(table.T).reshape(_VP, 2 * _L)
    out_dealt = _make_sc_kernel()(xs_flat, tab_packed)
    return out_dealt[:, _INV_DEAL]


# speedup vs baseline: 1.1480x; 1.1480x over previous
import functools

import jax
import jax.numpy as jnp
from jax import lax
from jax.experimental import pallas as pl
from jax.experimental.pallas import tpu as pltpu
from jax.experimental.pallas import tpu_sc as plsc

_B, _H, _D, _V = 16384, 50, 64, 1000000
_BR = 16384
_NB = 31
_S = _BR * _NB
_VP = 2 * _S

_NC, _NS, _L = 2, 16, 16
_NW = _NC * _NS
_BPW = _B // _NW
_C = 4
_CW = _C * _H
_NCH = _BPW // _C
_NBUF = 4
_INV = 1.0 / _H
_WIDX = _BPW * _H


def _pack_body(x1_ref, x2_ref, out_ref):
    x = jnp.concatenate([x1_ref[...], x2_ref[...]], axis=0)
    out_ref[...] = jnp.swapaxes(x, 0, 1)


def _pack(tab_t):
    return pl.pallas_call(
        _pack_body,
        out_shape=jax.ShapeDtypeStruct((_S, 2 * _D), jnp.float32),
        grid=(_NB,),
        in_specs=[
            pl.BlockSpec((_D, _BR), lambda g: (0, g)),
            pl.BlockSpec(
                (_D, _BR),
                lambda g: (0, jnp.minimum(g + _NB, (_V + _BR - 1) // _BR - 1)),
            ),
        ],
        out_specs=pl.BlockSpec((_BR, 2 * _D), lambda g: (g, 0)),
    )(tab_t, tab_t)


def _body(xs_hbm, tab_hbm, out_hbm, xs_v, rows_bufs, out_v, sems):
    wid = lax.axis_index("s") * _NC + lax.axis_index("c")
    base = wid * _BPW

    pltpu.sync_copy(xs_hbm.at[pl.ds(base * _H, _WIDX)], xs_v)

    @pl.loop(0, _WIDX, step=_L)
    def _remap(k):
        v = xs_v[pl.ds(k, _L)]
        ge = v >= _S
        xs_v[pl.ds(k, _L)] = 2 * v - jnp.where(ge, _VP - 1, 0)

    for b in range(_NBUF):
        pltpu.async_copy(
            tab_hbm.at[xs_v.at[pl.ds(b * _CW, _CW)]], rows_bufs[b], sems[b]
        )

    @pl.loop(0, _NCH, step=_NBUF)
    def _chunks(ci):
        for b in range(_NBUF):
            rows = rows_bufs[b]
            sem = sems[b]
            cur = ci + b
            pltpu.make_async_copy(
                tab_hbm.at[xs_v.at[pl.ds(cur * _CW, _CW)]], rows, sem
            ).wait()

            @pl.loop(0, _C)
            def _items(i, rows=rows, cur=cur):
                rowbase = i * _H
                z = jnp.zeros((_L,), jnp.float32)

                @pl.loop(0, _H, init_carry=(z, z, z, z), unroll=5)
                def _acc(j, carry, rows=rows, rowbase=rowbase):
                    a0, a1, a2, a3 = carry
                    rr = rowbase + j
                    return (
                        a0 + rows[rr, pl.ds(0, _L)],
                        a1 + rows[rr, pl.ds(_L, _L)],
                        a2 + rows[rr, pl.ds(2 * _L, _L)],
                        a3 + rows[rr, pl.ds(3 * _L, _L)],
                    )

                a0, a1, a2, a3 = _acc
                r0 = cur * _C + i
                out_v[r0, pl.ds(0, _L)] = a0 * _INV
                out_v[r0, pl.ds(_L, _L)] = a1 * _INV
                out_v[r0, pl.ds(2 * _L, _L)] = a2 * _INV
                out_v[r0, pl.ds(3 * _L, _L)] = a3 * _INV

            nxt = cur + _NBUF

            @pl.when(nxt < _NCH)
            def _fire(rows=rows, sem=sem, nxt=nxt):
                pltpu.async_copy(
                    tab_hbm.at[xs_v.at[pl.ds(nxt * _CW, _CW)]], rows, sem
                )

    pltpu.sync_copy(out_v, out_hbm.at[pl.ds(base, _BPW)])


@functools.cache
def _make_sc_kernel():
    mesh = plsc.VectorSubcoreMesh(
        core_axis_name="c", subcore_axis_name="s",
        num_cores=_NC, num_subcores=_NS,
    )

    def body(xs_hbm, tab_hbm, out_hbm, xs_v,
             r0, r1, r2, r3, out_v, s0, s1, s2, s3):
        _body(xs_hbm, tab_hbm, out_hbm, xs_v, (r0, r1, r2, r3), out_v,
              (s0, s1, s2, s3))

    return pl.kernel(
        body,
        out_type=jax.ShapeDtypeStruct((_B, _D), jnp.float32),
        mesh=mesh,
        scratch_types=[
            pltpu.VMEM((_WIDX,), jnp.int32),
            pltpu.VMEM((_CW, _D), jnp.float32),
            pltpu.VMEM((_CW, _D), jnp.float32),
            pltpu.VMEM((_CW, _D), jnp.float32),
            pltpu.VMEM((_CW, _D), jnp.float32),
            pltpu.VMEM((_BPW, _D), jnp.float32),
            pltpu.SemaphoreType.DMA,
            pltpu.SemaphoreType.DMA,
            pltpu.SemaphoreType.DMA,
            pltpu.SemaphoreType.DMA,
        ],
        compiler_params=pltpu.CompilerParams(use_tc_tiling_on_sc=False),
    )


def kernel(xs, table):
    xs_flat = xs.reshape(-1).astype(jnp.int32)
    tab_lin = _pack(table.T).reshape(_VP, _D)
    return _make_sc_kernel()(xs_flat, tab_lin)
